# baseline (speedup 1.0000x reference)
import functools

import jax
import jax.numpy as jnp
from jax import lax
from jax.experimental import pallas as pl
from jax.experimental.pallas import tpu as pltpu

_VAR0, _VAR1 = 0.1, 0.2
_THRESH = 0.5
_NEGPOS = 3


def _smooth_l1(x):
    ax = jnp.abs(x)
    return jnp.where(ax < 1.0, 0.5 * ax * ax, ax - 0.5)


def _k1_body(T, C, L, P,
             loc_ref, conf_ref, cdt_ref, pri_ref, tgt_ref, fs_ref, ft_ref,
             ll_ref, lcv_ref, ce_ref, pos_ref, kw_ref, w_ref,
             cms_ref, cmt_ref):
    b = pl.program_id(0)

    @pl.when(b == 0)
    def _init():
        ll_ref[0, 0] = 0.0

    pw = pri_ref[...]
    pcx, pcy = pw[0:1, :], pw[1:2, :]
    pwd, pht = pw[2:3, :], pw[3:4, :]
    px1 = pcx - pwd * 0.5
    py1 = pcy - pht * 0.5
    px2 = pcx + pwd * 0.5
    py2 = pcy + pht * 0.5
    area_p = (px2 - px1) * (py2 - py1)

    tgt = tgt_ref[0]

    iota_p = lax.broadcasted_iota(jnp.int32, (1, P), 1)

    bto = None
    bti = None
    bpidx = []
    for t in range(T):
        tx1, ty1, tx2, ty2 = tgt[t, 0], tgt[t, 1], tgt[t, 2], tgt[t, 3]
        iw = jnp.maximum(jnp.minimum(tx2, px2) - jnp.maximum(tx1, px1), 0.0)
        ih = jnp.maximum(jnp.minimum(ty2, py2) - jnp.maximum(ty1, py1), 0.0)
        inter = iw * ih
        area_t = (tx2 - tx1) * (ty2 - ty1)
        ov = inter / (area_t + area_p - inter)
        if t == 0:
            bto = ov
            bti = jnp.zeros((1, P), jnp.int32)
        else:
            upd = ov > bto
            bti = jnp.where(upd, t, bti)
            bto = jnp.where(upd, ov, bto)
        m = jnp.max(ov)
        bpidx.append(jnp.min(jnp.where(ov == m, iota_p, P)))
    for t in range(T):
        msk = iota_p == bpidx[t]
        bto = jnp.where(msk, 2.0, bto)
        bti = jnp.where(msk, t, bti)

    mx1 = my1 = mx2 = my2 = lbl = None
    for t in range(T):
        s = bti == t
        if t == 0:
            mx1, my1 = jnp.full((1, P), tgt[0, 0]), jnp.full((1, P), tgt[0, 1])
            mx2, my2 = jnp.full((1, P), tgt[0, 2]), jnp.full((1, P), tgt[0, 3])
            lbl = jnp.full((1, P), tgt[0, 4])
        else:
            mx1 = jnp.where(s, tgt[t, 0], mx1)
            my1 = jnp.where(s, tgt[t, 1], my1)
            mx2 = jnp.where(s, tgt[t, 2], mx2)
            my2 = jnp.where(s, tgt[t, 3], my2)
            lbl = jnp.where(s, tgt[t, 4], lbl)

    conf_t = jnp.where(bto < _THRESH, 0, (lbl + 1.0).astype(jnp.int32))
    pos = conf_t > 0
    posf = pos.astype(jnp.float32)
    npos_i = jnp.sum(conf_t > 0, dtype=jnp.int32)

    g_cx = ((mx1 + mx2) * 0.5 - pcx) / (_VAR0 * pwd)
    g_cy = ((my1 + my2) * 0.5 - pcy) / (_VAR0 * pht)
    g_w = jnp.log((mx2 - mx1) / pwd) / _VAR1
    g_h = jnp.log((my2 - my1) / pht) / _VAR1
    lw = jnp.transpose(loc_ref[0])
    sl = (_smooth_l1(lw[0:1, :] - g_cx) + _smooth_l1(lw[1:2, :] - g_cy)
          + _smooth_l1(lw[2:3, :] - g_w) + _smooth_l1(lw[3:4, :] - g_h))
    part_ll = jnp.sum(posf * sl)

    cs = jnp.transpose(conf_ref[0])
    ms = jnp.max(cs, axis=0, keepdims=True)
    es = jnp.exp(cs - ms)
    ss = jnp.sum(es, axis=0, keepdims=True)
    lss = jnp.log(ss)
    lse = lss + ms
    log_p = cs - ms - lss
    iota_c = lax.broadcasted_iota(jnp.int32, (C, P), 0)
    onehot = iota_c == conf_t
    gathered = jnp.sum(jnp.where(onehot, cs, 0.0), axis=0, keepdims=True)
    ce = lse - gathered
    lcv = jnp.where(pos, 0.0, ce)

    ct = jnp.transpose(cdt_ref[0])
    mt = jnp.max(ct, axis=0, keepdims=True)
    et = jnp.exp(ct - mt)
    st = jnp.sum(et, axis=0, keepdims=True)
    log_pt = ct - mt - jnp.log(st)
    p_t = et / st
    kl = p_t * log_pt - p_t * log_p
    kls = jnp.sum(kl, axis=0, keepdims=True)
    tqs = jnp.sum(-p_t * log_pt, axis=0, keepdims=True)
    w = (1.0 - jnp.exp(-kls - 2.0 * tqs)) ** 2

    cms_ref[...] = jnp.max(fs_ref[0], axis=0, keepdims=True).reshape(1, 1, L)
    cmt_ref[...] = jnp.max(ft_ref[0], axis=0, keepdims=True).reshape(1, 1, L)
    lcv_ref[...] = lcv.reshape(1, 1, P)
    ce_ref[...] = ce.reshape(1, 1, P)
    pos_ref[...] = posf.reshape(1, 1, P)
    kw_ref[...] = kls.reshape(1, 1, P)
    w_ref[...] = w.reshape(1, 1, P)
    ll_ref[0, 0] += part_ll


def _mine_body(P,
               lcv_ref, ce_ref, pos_ref, kw_ref, w_ref,
               lc_ref, ld_ref, np_ref, selw_ref, selb_ref):
    lcv = lcv_ref[:, 0, :]
    ce = ce_ref[:, 0, :]
    posf = pos_ref[:, 0, :]
    kls = kw_ref[:, 0, :]
    w = w_ref[:, 0, :]
    B = lcv.shape[0]

    npos = jnp.sum(posf, axis=1, keepdims=True)
    k = jnp.minimum(_NEGPOS * npos.astype(jnp.int32), P - 1)
    bits = lax.bitcast_convert_type(lcv, jnp.int32)
    iota_p = lax.broadcasted_iota(jnp.int32, (B, P), 1)

    def _bis_val(_, lohi):
        lo, hi = lohi
        mid = lo + (hi - lo) // 2
        ok = jnp.sum((bits >= mid).astype(jnp.int32), axis=1,
                     keepdims=True) >= k
        return jnp.where(ok, mid, lo), jnp.where(ok, hi, mid)

    tau, _ = lax.fori_loop(0, 31, _bis_val,
                           (jnp.full((B, 1), 0, jnp.int32),
                            jnp.full((B, 1), 0x7F800000, jnp.int32)))
    n_gt = jnp.sum((bits > tau).astype(jnp.int32), axis=1, keepdims=True)
    t_need = k - n_gt
    tie = bits == tau

    def _bis_idx(_, lohi):
        lo, hi = lohi
        mid = lo + (hi - lo) // 2
        ok = jnp.sum((tie & (iota_p <= mid)).astype(jnp.int32), axis=1,
                     keepdims=True) >= t_need
        return jnp.where(ok, lo, mid), jnp.where(ok, mid, hi)

    _, psi = lax.fori_loop(0, 14, _bis_idx,
                           (jnp.full((B, 1), -1, jnp.int32),
                            jnp.full((B, 1), P - 1, jnp.int32)))
    neg = (bits > tau) | (tie & (iota_p <= psi) & (t_need > 0))
    self_ = jnp.maximum(neg.astype(jnp.float32), posf)

    lc_ref[0, 0] = jnp.sum(self_ * ce)
    ld_ref[0, 0] = jnp.sum(self_ * kls * w)
    np_ref[0, 0] = jnp.sum(npos)
    selw_ref[...] = (self_ * w).reshape(B, 1, P)
    selb_ref[...] = self_.reshape(B, 1, P)


def _k3_body(fs_ref, ft_ref, coeff_ref, cms_ref, cmt_ref, selc_ref,
             out_ref, fsm_ref, ftm_ref):
    b = pl.program_id(0)
    j = pl.program_id(1)

    @pl.when((b == 0) & (j == 0))
    def _init():
        out_ref[0, 0] = 0.0
        selc = selc_ref[...]
        fsm_ref[0, 0] = jnp.max(jnp.where(selc > 0.0, cms_ref[...], -jnp.inf))
        ftm_ref[0, 0] = jnp.max(jnp.where(selc > 0.0, cmt_ref[...], -jnp.inf))

    rs = 1.0 / fsm_ref[0, 0]
    rt = 1.0 / ftm_ref[0, 0]
    d = fs_ref[0] * rs - ft_ref[0] * rt
    sl = _smooth_l1(d)
    colsum = jnp.sum(sl, axis=0, keepdims=True)
    out_ref[0, 0] += jnp.sum(coeff_ref[0] * colsum)


def kernel(loc_data, conf_data, loc_data_tch, conf_data_tch,
           feat_stu, feat_tch, priors, targets):
    del loc_data_tch
    B, P, C = conf_data.shape
    Cf, L = feat_stu.shape[1], feat_stu.shape[2]
    T = targets.shape[1]

    priw = priors.transpose(1, 0)

    body1 = functools.partial(_k1_body, T, C, L, P)
    sc = jax.ShapeDtypeStruct((1, 1), jnp.float32)
    smem_sc = pl.BlockSpec((1, 1), lambda b: (0, 0), memory_space=pltpu.SMEM)
    pvec = jax.ShapeDtypeStruct((B, 1, P), jnp.float32)
    pvec_spec = pl.BlockSpec((1, 1, P), lambda b: (b, 0, 0))
    lvec_spec = pl.BlockSpec((1, 1, L), lambda b: (b, 0, 0))
    feat_spec = pl.BlockSpec((1, Cf, L), lambda b: (b, 0, 0))

    lvec = jax.ShapeDtypeStruct((B, 1, L), jnp.float32)
    ll, lcv, cev, posv, kwv, wv, cms, cmt = pl.pallas_call(
        body1,
        grid=(B,),
        in_specs=[
            pl.BlockSpec((1, P, 4), lambda b: (b, 0, 0)),
            pl.BlockSpec((1, P, C), lambda b: (b, 0, 0)),
            pl.BlockSpec((1, P, C), lambda b: (b, 0, 0)),
            pl.BlockSpec((4, P), lambda b: (0, 0)),
            pl.BlockSpec((1, T, 5), lambda b: (b, 0, 0)),
            feat_spec,
            feat_spec,
        ],
        out_specs=[smem_sc, pvec_spec, pvec_spec, pvec_spec, pvec_spec,
                   pvec_spec, lvec_spec, lvec_spec],
        out_shape=[sc, pvec, pvec, pvec, pvec, pvec, lvec, lvec],
        compiler_params=pltpu.CompilerParams(
            dimension_semantics=("arbitrary",)),
    )(loc_data, conf_data, conf_data_tch, priw, targets,
      feat_stu, feat_tch)

    full_spec = pl.BlockSpec((B, 1, P), lambda: (0, 0, 0))
    smem_sc0 = pl.BlockSpec((1, 1), lambda: (0, 0), memory_space=pltpu.SMEM)
    lc, ld, npf, selw, selb = pl.pallas_call(
        functools.partial(_mine_body, P),
        grid=(),
        in_specs=[full_spec, full_spec, full_spec, full_spec, full_spec],
        out_specs=[smem_sc0, smem_sc0, smem_sc0, full_spec, full_spec],
        out_shape=[sc, sc, sc, pvec, pvec],
    )(lcv, cev, posv, kwv, wv)

    coeff = selw.reshape(B, 1, L, 2).sum(axis=3)
    selc = selb.reshape(B, 1, L, 2).max(axis=3)

    lvec_full = pl.BlockSpec((B, 1, L), lambda b, j: (0, 0, 0))
    half_feat = pl.BlockSpec((1, Cf // 2, L), lambda b, j: (b, j, 0))
    half_lvec = pl.BlockSpec((1, 1, L), lambda b, j: (b, 0, 0))
    smem_sc2 = pl.BlockSpec((1, 1), lambda b, j: (0, 0),
                            memory_space=pltpu.SMEM)
    (mim,) = pl.pallas_call(
        _k3_body,
        grid=(B, 2),
        in_specs=[half_feat, half_feat, half_lvec,
                  lvec_full, lvec_full, lvec_full],
        out_specs=[smem_sc2],
        out_shape=[sc],
        scratch_shapes=[pltpu.SMEM((1, 1), jnp.float32),
                        pltpu.SMEM((1, 1), jnp.float32)],
        compiler_params=pltpu.CompilerParams(
            dimension_semantics=("arbitrary", "arbitrary")),
    )(feat_stu, feat_tch, coeff, cms, cmt, selc)

    n = jnp.maximum(npf[0, 0], 1.0)
    return (ll[0, 0] / n, lc[0, 0] / n, ld[0, 0] / n, mim[0, 0] / n)


# speedup vs baseline: 1.0089x; 1.0089x over previous
import functools

import jax
import jax.numpy as jnp
from jax import lax
from jax.experimental import pallas as pl
from jax.experimental.pallas import tpu as pltpu

_VAR0, _VAR1 = 0.1, 0.2
_THRESH = 0.5
_NEGPOS = 3


def _smooth_l1(x):
    ax = jnp.abs(x)
    return jnp.where(ax < 1.0, 0.5 * ax * ax, ax - 0.5)


def _k1_body(T, C, L, P,
             loc_ref, conf_ref, cdt_ref, pri_ref, tgt_ref, fs_ref, ft_ref,
             ll_ref, lcv_ref, ce_ref, pos_ref, kw_ref, w_ref,
             cms_ref, cmt_ref):
    b = pl.program_id(0)

    @pl.when(b == 0)
    def _init():
        ll_ref[0, 0] = 0.0

    pw = pri_ref[...]
    pcx, pcy = pw[0:1, :], pw[1:2, :]
    pwd, pht = pw[2:3, :], pw[3:4, :]
    px1 = pcx - pwd * 0.5
    py1 = pcy - pht * 0.5
    px2 = pcx + pwd * 0.5
    py2 = pcy + pht * 0.5
    area_p = (px2 - px1) * (py2 - py1)

    tgt = tgt_ref[0]

    iota_p = lax.broadcasted_iota(jnp.int32, (1, P), 1)

    bto = None
    bti = None
    bpidx = []
    for t in range(T):
        tx1, ty1, tx2, ty2 = tgt[t, 0], tgt[t, 1], tgt[t, 2], tgt[t, 3]
        iw = jnp.maximum(jnp.minimum(tx2, px2) - jnp.maximum(tx1, px1), 0.0)
        ih = jnp.maximum(jnp.minimum(ty2, py2) - jnp.maximum(ty1, py1), 0.0)
        inter = iw * ih
        area_t = (tx2 - tx1) * (ty2 - ty1)
        ov = inter / (area_t + area_p - inter)
        if t == 0:
            bto = ov
            bti = jnp.zeros((1, P), jnp.int32)
        else:
            upd = ov > bto
            bti = jnp.where(upd, t, bti)
            bto = jnp.where(upd, ov, bto)
        m = jnp.max(ov)
        bpidx.append(jnp.min(jnp.where(ov == m, iota_p, P)))
    for t in range(T):
        msk = iota_p == bpidx[t]
        bto = jnp.where(msk, 2.0, bto)
        bti = jnp.where(msk, t, bti)

    mx1 = my1 = mx2 = my2 = lbl = None
    for t in range(T):
        s = bti == t
        if t == 0:
            mx1, my1 = jnp.full((1, P), tgt[0, 0]), jnp.full((1, P), tgt[0, 1])
            mx2, my2 = jnp.full((1, P), tgt[0, 2]), jnp.full((1, P), tgt[0, 3])
            lbl = jnp.full((1, P), tgt[0, 4])
        else:
            mx1 = jnp.where(s, tgt[t, 0], mx1)
            my1 = jnp.where(s, tgt[t, 1], my1)
            mx2 = jnp.where(s, tgt[t, 2], mx2)
            my2 = jnp.where(s, tgt[t, 3], my2)
            lbl = jnp.where(s, tgt[t, 4], lbl)

    conf_t = jnp.where(bto < _THRESH, 0, (lbl + 1.0).astype(jnp.int32))
    pos = conf_t > 0
    posf = pos.astype(jnp.float32)
    npos_i = jnp.sum(conf_t > 0, dtype=jnp.int32)

    g_cx = ((mx1 + mx2) * 0.5 - pcx) / (_VAR0 * pwd)
    g_cy = ((my1 + my2) * 0.5 - pcy) / (_VAR0 * pht)
    g_w = jnp.log((mx2 - mx1) / pwd) / _VAR1
    g_h = jnp.log((my2 - my1) / pht) / _VAR1
    lw = jnp.transpose(loc_ref[0])
    sl = (_smooth_l1(lw[0:1, :] - g_cx) + _smooth_l1(lw[1:2, :] - g_cy)
          + _smooth_l1(lw[2:3, :] - g_w) + _smooth_l1(lw[3:4, :] - g_h))
    part_ll = jnp.sum(posf * sl)

    cs = jnp.transpose(conf_ref[0])
    ms = jnp.max(cs, axis=0, keepdims=True)
    es = jnp.exp(cs - ms)
    ss = jnp.sum(es, axis=0, keepdims=True)
    lss = jnp.log(ss)
    lse = lss + ms
    log_p = cs - ms - lss
    iota_c = lax.broadcasted_iota(jnp.int32, (C, P), 0)
    onehot = iota_c == conf_t
    gathered = jnp.sum(jnp.where(onehot, cs, 0.0), axis=0, keepdims=True)
    ce = lse - gathered
    lcv = jnp.where(pos, 0.0, ce)

    ct = jnp.transpose(cdt_ref[0])
    mt = jnp.max(ct, axis=0, keepdims=True)
    et = jnp.exp(ct - mt)
    st = jnp.sum(et, axis=0, keepdims=True)
    log_pt = ct - mt - jnp.log(st)
    p_t = et / st
    kl = p_t * log_pt - p_t * log_p
    kls = jnp.sum(kl, axis=0, keepdims=True)
    tqs = jnp.sum(-p_t * log_pt, axis=0, keepdims=True)
    w = (1.0 - jnp.exp(-kls - 2.0 * tqs)) ** 2

    cms_ref[...] = jnp.max(fs_ref[0], axis=0, keepdims=True).reshape(1, 1, L)
    cmt_ref[...] = jnp.max(ft_ref[0], axis=0, keepdims=True).reshape(1, 1, L)
    lcv_ref[...] = lcv.reshape(1, 1, P)
    ce_ref[...] = ce.reshape(1, 1, P)
    pos_ref[...] = posf.reshape(1, 1, P)
    kw_ref[...] = kls.reshape(1, 1, P)
    w_ref[...] = w.reshape(1, 1, P)
    ll_ref[0, 0] += part_ll


def _mine_body(P,
               lcv_ref, ce_ref, pos_ref, kw_ref, w_ref,
               lc_ref, ld_ref, np_ref, selw_ref, selb_ref):
    lcv = lcv_ref[:, 0, :]
    ce = ce_ref[:, 0, :]
    posf = pos_ref[:, 0, :]
    kls = kw_ref[:, 0, :]
    w = w_ref[:, 0, :]
    B = lcv.shape[0]

    npos = jnp.sum(posf, axis=1, keepdims=True)
    k = jnp.minimum(_NEGPOS * npos.astype(jnp.int32), P - 1)
    bits = lax.bitcast_convert_type(lcv, jnp.int32)
    iota_p = lax.broadcasted_iota(jnp.int32, (B, P), 1)

    def _bis_val(_, lohi):
        lo, hi = lohi
        mid = lo + (hi - lo) // 2
        ok = jnp.sum((bits >= mid).astype(jnp.int32), axis=1,
                     keepdims=True) >= k
        return jnp.where(ok, mid, lo), jnp.where(ok, hi, mid)

    tau, _ = lax.fori_loop(0, 31, _bis_val,
                           (jnp.full((B, 1), 0, jnp.int32),
                            jnp.full((B, 1), 0x7F800000, jnp.int32)))
    n_gt = jnp.sum((bits > tau).astype(jnp.int32), axis=1, keepdims=True)
    t_need = k - n_gt
    tie = bits == tau

    def _bis_idx(_, lohi):
        lo, hi = lohi
        mid = lo + (hi - lo) // 2
        ok = jnp.sum((tie & (iota_p <= mid)).astype(jnp.int32), axis=1,
                     keepdims=True) >= t_need
        return jnp.where(ok, lo, mid), jnp.where(ok, mid, hi)

    _, psi = lax.fori_loop(0, 14, _bis_idx,
                           (jnp.full((B, 1), -1, jnp.int32),
                            jnp.full((B, 1), P - 1, jnp.int32)))
    neg = (bits > tau) | (tie & (iota_p <= psi) & (t_need > 0))
    self_ = jnp.maximum(neg.astype(jnp.float32), posf)

    lc_ref[0, 0] = jnp.sum(self_ * ce)
    ld_ref[0, 0] = jnp.sum(self_ * kls * w)
    np_ref[0, 0] = jnp.sum(npos)
    selw_ref[...] = (self_ * w).reshape(B, 1, P)
    selb_ref[...] = self_.reshape(B, 1, P)


def _k3_body(fs_ref, ft_ref, coeff_ref, cms_ref, cmt_ref, selc_ref,
             out_ref, fsm_ref, ftm_ref):
    b = pl.program_id(0)

    @pl.when(b == 0)
    def _init():
        out_ref[0, 0] = 0.0
        selc = selc_ref[...]
        fsm_ref[0, 0] = jnp.max(jnp.where(selc > 0.0, cms_ref[...], -jnp.inf))
        ftm_ref[0, 0] = jnp.max(jnp.where(selc > 0.0, cmt_ref[...], -jnp.inf))

    rs = 1.0 / fsm_ref[0, 0]
    rt = 1.0 / ftm_ref[0, 0]
    d = fs_ref[0] * rs - ft_ref[0] * rt
    sl = _smooth_l1(d)
    colsum = jnp.sum(sl, axis=0, keepdims=True)
    out_ref[0, 0] += jnp.sum(coeff_ref[0] * colsum)


def kernel(loc_data, conf_data, loc_data_tch, conf_data_tch,
           feat_stu, feat_tch, priors, targets):
    del loc_data_tch
    B, P, C = conf_data.shape
    Cf, L = feat_stu.shape[1], feat_stu.shape[2]
    T = targets.shape[1]

    priw = priors.transpose(1, 0)

    body1 = functools.partial(_k1_body, T, C, L, P)
    sc = jax.ShapeDtypeStruct((1, 1), jnp.float32)
    smem_sc = pl.BlockSpec((1, 1), lambda b: (0, 0), memory_space=pltpu.SMEM)
    pvec = jax.ShapeDtypeStruct((B, 1, P), jnp.float32)
    pvec_spec = pl.BlockSpec((1, 1, P), lambda b: (b, 0, 0))
    lvec_spec = pl.BlockSpec((1, 1, L), lambda b: (b, 0, 0))
    feat_spec = pl.BlockSpec((1, Cf, L), lambda b: (b, 0, 0))

    lvec = jax.ShapeDtypeStruct((B, 1, L), jnp.float32)
    ll, lcv, cev, posv, kwv, wv, cms, cmt = pl.pallas_call(
        body1,
        grid=(B,),
        in_specs=[
            pl.BlockSpec((1, P, 4), lambda b: (b, 0, 0)),
            pl.BlockSpec((1, P, C), lambda b: (b, 0, 0)),
            pl.BlockSpec((1, P, C), lambda b: (b, 0, 0)),
            pl.BlockSpec((4, P), lambda b: (0, 0)),
            pl.BlockSpec((1, T, 5), lambda b: (b, 0, 0)),
            feat_spec,
            feat_spec,
        ],
        out_specs=[smem_sc, pvec_spec, pvec_spec, pvec_spec, pvec_spec,
                   pvec_spec, lvec_spec, lvec_spec],
        out_shape=[sc, pvec, pvec, pvec, pvec, pvec, lvec, lvec],
        compiler_params=pltpu.CompilerParams(
            dimension_semantics=("arbitrary",)),
    )(loc_data, conf_data, conf_data_tch, priw, targets,
      feat_stu, feat_tch)

    full_spec = pl.BlockSpec((B, 1, P), lambda: (0, 0, 0))
    smem_sc0 = pl.BlockSpec((1, 1), lambda: (0, 0), memory_space=pltpu.SMEM)
    lc, ld, npf, selw, selb = pl.pallas_call(
        functools.partial(_mine_body, P),
        grid=(),
        in_specs=[full_spec, full_spec, full_spec, full_spec, full_spec],
        out_specs=[smem_sc0, smem_sc0, smem_sc0, full_spec, full_spec],
        out_shape=[sc, sc, sc, pvec, pvec],
    )(lcv, cev, posv, kwv, wv)

    coeff = selw.reshape(B, 1, L, 2).sum(axis=3)
    selc = selb.reshape(B, 1, L, 2).max(axis=3)

    lvec_full = pl.BlockSpec((B, 1, L), lambda b: (0, 0, 0))
    (mim,) = pl.pallas_call(
        _k3_body,
        grid=(B,),
        in_specs=[feat_spec, feat_spec, lvec_spec,
                  lvec_full, lvec_full, lvec_full],
        out_specs=[smem_sc],
        out_shape=[sc],
        scratch_shapes=[pltpu.SMEM((1, 1), jnp.float32),
                        pltpu.SMEM((1, 1), jnp.float32)],
        compiler_params=pltpu.CompilerParams(
            dimension_semantics=("arbitrary",)),
    )(feat_stu, feat_tch, coeff, cms, cmt, selc)

    n = jnp.maximum(npf[0, 0], 1.0)
    return (ll[0, 0] / n, lc[0, 0] / n, ld[0, 0] / n, mim[0, 0] / n)
